# baseline (speedup 1.0000x reference)
import functools

import jax
import jax.numpy as jnp
from jax.experimental import pallas as pl
from jax.experimental.pallas import tpu as pltpu


def _chamfer_body(xp_ref, x2_ref, yp_ref, y2_ref, rowout_ref, colout_ref,
                  rowacc_ref, *, num_mblocks):
    j = pl.program_id(1)
    xp = xp_ref[0].astype(jnp.float32)
    yp = yp_ref[0].astype(jnp.float32)
    xsq = x2_ref[0]
    ysq = y2_ref[0]

    z = xp[:, 0:1] * yp[0:1, :] + xp[:, 1:2] * yp[1:2, :] \
        + xp[:, 2:3] * yp[2:3, :] + ysq
    w = z - ysq + xsq

    colout_ref[0, 0, :] = jnp.maximum(jnp.min(w, axis=0) + ysq[0, :], 0.0)

    @pl.when(j == 0)
    def _init():
        rowacc_ref[...] = z

    @pl.when(j > 0)
    def _acc():
        rowacc_ref[...] = jnp.minimum(rowacc_ref[...], z)

    @pl.when(j == num_mblocks - 1)
    def _finish():
        rowmin = jnp.min(rowacc_ref[...], axis=1)
        rowout_ref[0, 0, :] = jnp.maximum(rowmin + xsq[:, 0], 0.0)


def kernel(x, y):
    B, N, D = x.shape
    M = y.shape[1]
    BM = 512
    num_mblocks = M // BM

    x2 = jnp.sum(x * x, axis=-1)[..., None]
    y2 = jnp.sum(y * y, axis=-1)[:, None, :]
    xp = (-2.0 * x).astype(jnp.bfloat16)
    yp = jnp.swapaxes(y, 1, 2).astype(jnp.bfloat16)

    rowmin, colmin = pl.pallas_call(
        functools.partial(_chamfer_body, num_mblocks=num_mblocks),
        grid=(B, num_mblocks),
        in_specs=[
            pl.BlockSpec((1, N, 3), lambda b, j: (b, 0, 0)),
            pl.BlockSpec((1, N, 1), lambda b, j: (b, 0, 0)),
            pl.BlockSpec((1, 3, BM), lambda b, j: (b, 0, j)),
            pl.BlockSpec((1, 1, BM), lambda b, j: (b, 0, j)),
        ],
        out_specs=[
            pl.BlockSpec((1, 1, N), lambda b, j: (b, 0, 0)),
            pl.BlockSpec((1, 1, BM), lambda b, j: (b, 0, j)),
        ],
        out_shape=[
            jax.ShapeDtypeStruct((B, 1, N), jnp.float32),
            jax.ShapeDtypeStruct((B, 1, M), jnp.float32),
        ],
        scratch_shapes=[pltpu.VMEM((N, BM), jnp.float32)],
        compiler_params=pltpu.CompilerParams(
            dimension_semantics=("parallel", "arbitrary")),
    )(xp, x2, yp, y2)

    return jnp.mean(rowmin) + jnp.mean(colmin)


# speedup vs baseline: 1.0354x; 1.0354x over previous
import functools

import jax
import jax.numpy as jnp
from jax.experimental import pallas as pl
from jax.experimental.pallas import tpu as pltpu


def _chamfer_body(xp_ref, x2_ref, yp_ref, y2_ref, rowout_ref, colout_ref,
                  rowacc_ref, *, num_mblocks):
    j = pl.program_id(1)
    xp = xp_ref[0].astype(jnp.float32)
    yp = yp_ref[0].astype(jnp.float32)
    xsq = x2_ref[0]
    ysq = y2_ref[0]

    w = xp[:, 0:1] * yp[0:1, :] + xp[:, 1:2] * yp[1:2, :] \
        + xp[:, 2:3] * yp[2:3, :] + ysq + xsq

    colout_ref[0, 0, :] = jnp.maximum(jnp.min(w, axis=0), 0.0)

    @pl.when(j == 0)
    def _init():
        rowacc_ref[...] = w

    @pl.when(j > 0)
    def _acc():
        rowacc_ref[...] = jnp.minimum(rowacc_ref[...], w)

    @pl.when(j == num_mblocks - 1)
    def _finish():
        rowmin = jnp.min(rowacc_ref[...], axis=1)
        rowout_ref[0, 0, :] = jnp.maximum(rowmin, 0.0)


def kernel(x, y):
    B, N, D = x.shape
    M = y.shape[1]
    BM = 512
    num_mblocks = M // BM

    x2 = jnp.sum(x * x, axis=-1)[..., None]
    y2 = jnp.sum(y * y, axis=-1)[:, None, :]
    xp = (-2.0 * x).astype(jnp.bfloat16)
    yp = jnp.swapaxes(y, 1, 2).astype(jnp.bfloat16)

    rowmin, colmin = pl.pallas_call(
        functools.partial(_chamfer_body, num_mblocks=num_mblocks),
        grid=(B, num_mblocks),
        in_specs=[
            pl.BlockSpec((1, N, 3), lambda b, j: (b, 0, 0)),
            pl.BlockSpec((1, N, 1), lambda b, j: (b, 0, 0)),
            pl.BlockSpec((1, 3, BM), lambda b, j: (b, 0, j)),
            pl.BlockSpec((1, 1, BM), lambda b, j: (b, 0, j)),
        ],
        out_specs=[
            pl.BlockSpec((1, 1, N), lambda b, j: (b, 0, 0)),
            pl.BlockSpec((1, 1, BM), lambda b, j: (b, 0, j)),
        ],
        out_shape=[
            jax.ShapeDtypeStruct((B, 1, N), jnp.float32),
            jax.ShapeDtypeStruct((B, 1, M), jnp.float32),
        ],
        scratch_shapes=[pltpu.VMEM((N, BM), jnp.float32)],
        compiler_params=pltpu.CompilerParams(
            dimension_semantics=("parallel", "arbitrary")),
    )(xp, x2, yp, y2)

    return jnp.mean(rowmin) + jnp.mean(colmin)


# speedup vs baseline: 2.2567x; 2.1796x over previous
import functools

import jax
import jax.numpy as jnp
from jax.experimental import pallas as pl
from jax.experimental.pallas import tpu as pltpu

_BM = 512
_BH = 256


def _dot_dims():
    return (((0,), (0,)), ((), ()))


def _tree_colmin(dg):
    n = dg.shape[0]
    while n > 256:
        n //= 4
        dg = jnp.minimum(jnp.minimum(dg[:n], dg[n:2 * n]),
                         jnp.minimum(dg[2 * n:3 * n], dg[3 * n:]))
    return jnp.min(dg, axis=0)


def _reduce_half(d, colout_ref, base):
    g0 = d[:, :128]
    g1 = d[:, 128:]
    colout_ref[0, 0, base:base + 128] = jnp.maximum(_tree_colmin(g0), 0.0)
    colout_ref[0, 0, base + 128:base + 256] = jnp.maximum(
        _tree_colmin(g1), 0.0)
    return jnp.minimum(g0, g1)


def _chamfer_body(xpa_ref, ypa_ref, rowout_ref, colout_ref,
                  z0_ref, z1_ref, rowacc_ref, *, num_mblocks):
    j = pl.program_id(1)
    xpa = xpa_ref[0]
    ypa = ypa_ref[0]

    f1 = _reduce_half(z1_ref[...], colout_ref, _BH)

    z0_ref[...] = jax.lax.dot_general(xpa, ypa[:, :_BH], _dot_dims(),
                                      preferred_element_type=jnp.float32)
    f0 = _reduce_half(z0_ref[...], colout_ref, 0)

    acc = jnp.minimum(rowacc_ref[...], jnp.minimum(f0, f1))
    rowacc_ref[...] = jnp.where(j == 0, f0, acc)

    z1_ref[...] = jax.lax.dot_general(xpa, ypa[:, _BH:], _dot_dims(),
                                      preferred_element_type=jnp.float32)

    @pl.when(j == num_mblocks)
    def _finish():
        rowmin = jnp.min(rowacc_ref[...], axis=1, keepdims=True)
        rowout_ref[0, :, :] = jnp.maximum(rowmin, 0.0)


def _hi_lo(v):
    hi = jax.lax.bitcast_convert_type(
        jax.lax.bitcast_convert_type(v, jnp.uint32) & jnp.uint32(0xFFFF0000),
        jnp.float32)
    return hi, v - hi


def kernel(x, y):
    B, N, D = x.shape
    M = y.shape[1]
    num_mblocks = M // _BM

    x2 = jnp.sum(x * x, axis=-1)[:, None, :]
    y2 = jnp.sum(y * y, axis=-1)[:, None, :]
    x2hi, x2lo = _hi_lo(x2)
    y2hi, y2lo = _hi_lo(y2)
    ones_n = jnp.ones((B, 1, N), jnp.float32)
    ones_m = jnp.ones((B, 1, M), jnp.float32)
    xpa = jnp.concatenate(
        [jnp.swapaxes(-2.0 * x, 1, 2), x2hi, x2lo, ones_n, ones_n],
        axis=1).astype(jnp.bfloat16)
    ypa = jnp.concatenate(
        [jnp.swapaxes(y, 1, 2), ones_m, ones_m, y2hi, y2lo],
        axis=1).astype(jnp.bfloat16)

    nj = num_mblocks + 1
    yidx = lambda b, j: (b, 0, jnp.minimum(j, num_mblocks - 1))

    rowmin, colraw = pl.pallas_call(
        functools.partial(_chamfer_body, num_mblocks=num_mblocks),
        grid=(B, nj),
        in_specs=[
            pl.BlockSpec((1, 7, N), lambda b, j: (b, 0, 0)),
            pl.BlockSpec((1, 7, _BM), yidx),
        ],
        out_specs=[
            pl.BlockSpec((1, N, 1), lambda b, j: (b, 0, 0)),
            pl.BlockSpec((1, 1, _BM), lambda b, j: (b, 0, j)),
        ],
        out_shape=[
            jax.ShapeDtypeStruct((B, N, 1), jnp.float32),
            jax.ShapeDtypeStruct((B, 1, nj * _BM), jnp.float32),
        ],
        scratch_shapes=[
            pltpu.VMEM((N, _BH), jnp.float32),
            pltpu.VMEM((N, _BH), jnp.float32),
            pltpu.VMEM((N, 128), jnp.float32),
        ],
        compiler_params=pltpu.CompilerParams(
            dimension_semantics=("parallel", "arbitrary")),
    )(xpa, ypa)

    c = colraw.reshape(B, nj, 2, _BH)
    sub0 = c[:, :num_mblocks, 0, :]
    sub1 = c[:, 1:, 1, :]
    colmin = jnp.stack([sub0, sub1], axis=2).reshape(B, M)

    return jnp.mean(rowmin) + jnp.mean(colmin)


# speedup vs baseline: 2.2723x; 1.0069x over previous
import functools

import jax
import jax.numpy as jnp
from jax.experimental import pallas as pl
from jax.experimental.pallas import tpu as pltpu

_BM = 1024
_BH = 512


def _dot_dims():
    return (((0,), (0,)), ((), ()))


def _tree_colmin(dg):
    n = dg.shape[0]
    while n > 256:
        n //= 4
        dg = jnp.minimum(jnp.minimum(dg[:n], dg[n:2 * n]),
                         jnp.minimum(dg[2 * n:3 * n], dg[3 * n:]))
    return jnp.min(dg, axis=0)


def _reduce_half(d, colout_ref, base):
    folds = []
    for g in range(_BH // 128):
        dg = d[:, g * 128:(g + 1) * 128]
        colout_ref[0, 0, base + g * 128:base + (g + 1) * 128] = \
            jnp.maximum(_tree_colmin(dg), 0.0)
        folds.append(dg)
    while len(folds) > 1:
        folds = [jnp.minimum(folds[i], folds[i + 1])
                 for i in range(0, len(folds), 2)]
    return folds[0]


def _chamfer_body(xpa_ref, ypa_ref, rowout_ref, colout_ref,
                  z0_ref, z1_ref, rowacc_ref, *, num_mblocks):
    j = pl.program_id(1)
    xpa = xpa_ref[0]
    ypa = ypa_ref[0]

    f1 = _reduce_half(z1_ref[...], colout_ref, _BH)

    z0_ref[...] = jax.lax.dot_general(xpa, ypa[:, :_BH], _dot_dims(),
                                      preferred_element_type=jnp.float32)
    f0 = _reduce_half(z0_ref[...], colout_ref, 0)

    acc = jnp.minimum(rowacc_ref[...], jnp.minimum(f0, f1))
    rowacc_ref[...] = jnp.where(j == 0, f0, acc)

    z1_ref[...] = jax.lax.dot_general(xpa, ypa[:, _BH:], _dot_dims(),
                                      preferred_element_type=jnp.float32)

    @pl.when(j == num_mblocks)
    def _finish():
        rowmin = jnp.min(rowacc_ref[...], axis=1, keepdims=True)
        rowout_ref[0, :, :] = jnp.maximum(rowmin, 0.0)


def _hi_lo(v):
    hi = jax.lax.bitcast_convert_type(
        jax.lax.bitcast_convert_type(v, jnp.uint32) & jnp.uint32(0xFFFF0000),
        jnp.float32)
    return hi, v - hi


def kernel(x, y):
    B, N, D = x.shape
    M = y.shape[1]
    num_mblocks = M // _BM

    x2 = jnp.sum(x * x, axis=-1)[:, None, :]
    y2 = jnp.sum(y * y, axis=-1)[:, None, :]
    x2hi, x2lo = _hi_lo(x2)
    y2hi, y2lo = _hi_lo(y2)
    ones_n = jnp.ones((B, 1, N), jnp.float32)
    ones_m = jnp.ones((B, 1, M), jnp.float32)
    xpa = jnp.concatenate(
        [jnp.swapaxes(-2.0 * x, 1, 2), x2hi, x2lo, ones_n, ones_n],
        axis=1).astype(jnp.bfloat16)
    ypa = jnp.concatenate(
        [jnp.swapaxes(y, 1, 2), ones_m, ones_m, y2hi, y2lo],
        axis=1).astype(jnp.bfloat16)

    nj = num_mblocks + 1
    yidx = lambda b, j: (b, 0, jnp.minimum(j, num_mblocks - 1))

    rowmin, colraw = pl.pallas_call(
        functools.partial(_chamfer_body, num_mblocks=num_mblocks),
        grid=(B, nj),
        in_specs=[
            pl.BlockSpec((1, 7, N), lambda b, j: (b, 0, 0)),
            pl.BlockSpec((1, 7, _BM), yidx),
        ],
        out_specs=[
            pl.BlockSpec((1, N, 1), lambda b, j: (b, 0, 0)),
            pl.BlockSpec((1, 1, _BM), lambda b, j: (b, 0, j)),
        ],
        out_shape=[
            jax.ShapeDtypeStruct((B, N, 1), jnp.float32),
            jax.ShapeDtypeStruct((B, 1, nj * _BM), jnp.float32),
        ],
        scratch_shapes=[
            pltpu.VMEM((N, _BH), jnp.float32),
            pltpu.VMEM((N, _BH), jnp.float32),
            pltpu.VMEM((N, 128), jnp.float32),
        ],
        compiler_params=pltpu.CompilerParams(
            dimension_semantics=("parallel", "arbitrary")),
    )(xpa, ypa)

    c = colraw.reshape(B, nj, 2, _BH)
    sub0 = c[:, :num_mblocks, 0, :]
    sub1 = c[:, 1:, 1, :]
    colmin = jnp.stack([sub0, sub1], axis=2).reshape(B, M)

    return jnp.mean(rowmin) + jnp.mean(colmin)


# speedup vs baseline: 2.4759x; 1.0896x over previous
import functools

import jax
import jax.numpy as jnp
from jax.experimental import pallas as pl
from jax.experimental.pallas import tpu as pltpu

_BM = 1024
_BH = 512


def _dot_dims():
    return (((0,), (0,)), ((), ()))


def _tree_colmin(dg):
    n = dg.shape[0]
    while n > 256:
        n //= 4
        dg = jnp.minimum(jnp.minimum(dg[:n], dg[n:2 * n]),
                         jnp.minimum(dg[2 * n:3 * n], dg[3 * n:]))
    return jnp.min(dg, axis=0, keepdims=True)


def _reduce_half(d):
    folds = []
    csum = None
    for g in range(_BH // 128):
        dg = d[:, g * 128:(g + 1) * 128]
        cm = jnp.maximum(_tree_colmin(dg), 0.0)
        csum = cm if csum is None else csum + cm
        folds.append(dg)
    while len(folds) > 1:
        folds = [jnp.minimum(folds[i], folds[i + 1])
                 for i in range(0, len(folds), 2)]
    return folds[0], csum


def _chamfer_body(xpa_ref, ypa_ref, rowsum_ref, colsum_ref,
                  z0_ref, z1_ref, rowacc_ref, csum_ref, *, num_mblocks):
    j = pl.program_id(1)
    xpa = xpa_ref[0]
    ypa = ypa_ref[0]

    f1, s1 = _reduce_half(z1_ref[...])

    z0_ref[...] = jax.lax.dot_general(xpa, ypa[:, :_BH], _dot_dims(),
                                      preferred_element_type=jnp.float32)
    f0, s0 = _reduce_half(z0_ref[...])

    acc = jnp.minimum(rowacc_ref[...], jnp.minimum(f0, f1))
    rowacc_ref[...] = jnp.where(j == 0, f0, acc)

    upd = csum_ref[...] + jnp.where(j == num_mblocks, 0.0, s0) + s1
    csum_ref[...] = jnp.where(j == 0, s0, upd)

    z1_ref[...] = jax.lax.dot_general(xpa, ypa[:, _BH:], _dot_dims(),
                                      preferred_element_type=jnp.float32)

    @pl.when(j == num_mblocks)
    def _finish():
        rowmin = jnp.min(rowacc_ref[...], axis=1, keepdims=True)
        rowclamped = jnp.maximum(rowmin, 0.0)
        rowsum_ref[0, :, :] = jnp.sum(rowclamped, axis=0, keepdims=True)
        colsum_ref[0, :, :] = jnp.sum(csum_ref[...], axis=1, keepdims=True)


def _hi_lo(v):
    hi = jax.lax.bitcast_convert_type(
        jax.lax.bitcast_convert_type(v, jnp.uint32) & jnp.uint32(0xFFFF0000),
        jnp.float32)
    return hi, v - hi


def kernel(x, y):
    B, N, D = x.shape
    M = y.shape[1]
    num_mblocks = M // _BM

    x2 = jnp.sum(x * x, axis=-1)[:, None, :]
    y2 = jnp.sum(y * y, axis=-1)[:, None, :]
    x2hi, x2lo = _hi_lo(x2)
    y2hi, y2lo = _hi_lo(y2)
    ones_n = jnp.ones((B, 1, N), jnp.float32)
    ones_m = jnp.ones((B, 1, M), jnp.float32)
    xpa = jnp.concatenate(
        [jnp.swapaxes(-2.0 * x, 1, 2), x2hi, x2lo, ones_n, ones_n],
        axis=1).astype(jnp.bfloat16)
    ypa = jnp.concatenate(
        [jnp.swapaxes(y, 1, 2), ones_m, ones_m, y2hi, y2lo],
        axis=1).astype(jnp.bfloat16)

    nj = num_mblocks + 1
    yidx = lambda b, j: (b, 0, jnp.minimum(j, num_mblocks - 1))

    rowsums, colsums = pl.pallas_call(
        functools.partial(_chamfer_body, num_mblocks=num_mblocks),
        grid=(B, nj),
        in_specs=[
            pl.BlockSpec((1, 7, N), lambda b, j: (b, 0, 0)),
            pl.BlockSpec((1, 7, _BM), yidx),
        ],
        out_specs=[
            pl.BlockSpec((1, 1, 1), lambda b, j: (b, 0, 0)),
            pl.BlockSpec((1, 1, 1), lambda b, j: (b, 0, 0)),
        ],
        out_shape=[
            jax.ShapeDtypeStruct((B, 1, 1), jnp.float32),
            jax.ShapeDtypeStruct((B, 1, 1), jnp.float32),
        ],
        scratch_shapes=[
            pltpu.VMEM((N, _BH), jnp.float32),
            pltpu.VMEM((N, _BH), jnp.float32),
            pltpu.VMEM((N, 128), jnp.float32),
            pltpu.VMEM((1, 128), jnp.float32),
        ],
        compiler_params=pltpu.CompilerParams(
            dimension_semantics=("parallel", "arbitrary")),
    )(xpa, ypa)

    return (jnp.sum(rowsums) / (B * N)) + (jnp.sum(colsums) / (B * M))


# speedup vs baseline: 2.9189x; 1.1789x over previous
import functools

import jax
import jax.numpy as jnp
from jax.experimental import pallas as pl
from jax.experimental.pallas import tpu as pltpu

_BM = 1024
_BH = 512


def _dot_dims():
    return (((0,), (0,)), ((), ()))


def _tree_colmin(dg):
    n = dg.shape[0]
    while n > 256:
        n //= 4
        dg = jnp.minimum(jnp.minimum(dg[:n], dg[n:2 * n]),
                         jnp.minimum(dg[2 * n:3 * n], dg[3 * n:]))
    return jnp.min(dg, axis=0, keepdims=True)


def _reduce_half(d):
    folds = []
    csum = None
    for g in range(_BH // 128):
        dg = d[:, g * 128:(g + 1) * 128]
        cm = jnp.maximum(_tree_colmin(dg), 0.0)
        csum = cm if csum is None else csum + cm
        folds.append(dg)
    while len(folds) > 1:
        folds = [jnp.minimum(folds[i], folds[i + 1])
                 for i in range(0, len(folds), 2)]
    return folds[0], csum


def _chamfer_body(xpa_ref, ypa_ref, rowsum_ref, colsum_ref,
                  z0_ref, z1_ref, rowacc_ref, csum_ref, *, num_mblocks):
    j = pl.program_id(1)
    xpa = xpa_ref[0]
    ypa = ypa_ref[0]

    f1, s1 = _reduce_half(z1_ref[...])

    z0_ref[...] = jax.lax.dot_general(xpa, ypa[:, :_BH], _dot_dims(),
                                      preferred_element_type=jnp.float32)
    f0, s0 = _reduce_half(z0_ref[...])

    acc = jnp.minimum(rowacc_ref[...], jnp.minimum(f0, f1))
    rowacc_ref[...] = jnp.where(j == 0, f0, acc)

    csum_ref[...] = jnp.where(j == 0, s0, csum_ref[...] + s0 + s1)

    z1_ref[...] = jax.lax.dot_general(xpa, ypa[:, _BH:], _dot_dims(),
                                      preferred_element_type=jnp.float32)

    @pl.when(j == num_mblocks - 1)
    def _finish():
        f1b, s1b = _reduce_half(z1_ref[...])
        rowall = jnp.minimum(rowacc_ref[...], f1b)
        rowmin = jnp.min(rowall, axis=1, keepdims=True)
        rowclamped = jnp.maximum(rowmin, 0.0)
        rowsum_ref[0, :, :] = jnp.sum(rowclamped, axis=0, keepdims=True)
        colsum_ref[0, :, :] = jnp.sum(csum_ref[...] + s1b, axis=1,
                                      keepdims=True)


def _hi_lo(v):
    hi = jax.lax.bitcast_convert_type(
        jax.lax.bitcast_convert_type(v, jnp.uint32) & jnp.uint32(0xFFFF0000),
        jnp.float32)
    return hi, v - hi


def kernel(x, y):
    B, N, D = x.shape
    M = y.shape[1]
    num_mblocks = M // _BM

    x2 = jnp.sum(x * x, axis=-1)[:, None, :]
    y2 = jnp.sum(y * y, axis=-1)[:, None, :]
    x2hi, x2lo = _hi_lo(x2)
    y2hi, y2lo = _hi_lo(y2)
    ones_n = jnp.ones((B, 1, N), jnp.float32)
    ones_m = jnp.ones((B, 1, M), jnp.float32)
    xpa = jnp.concatenate(
        [jnp.swapaxes(-2.0 * x, 1, 2), x2hi, x2lo, ones_n, ones_n],
        axis=1).astype(jnp.bfloat16)
    ypa = jnp.concatenate(
        [jnp.swapaxes(y, 1, 2), ones_m, ones_m, y2hi, y2lo],
        axis=1).astype(jnp.bfloat16)

    nj = num_mblocks
    yidx = lambda b, j: (b, 0, j)

    rowsums, colsums = pl.pallas_call(
        functools.partial(_chamfer_body, num_mblocks=num_mblocks),
        grid=(B, nj),
        in_specs=[
            pl.BlockSpec((1, 7, N), lambda b, j: (b, 0, 0)),
            pl.BlockSpec((1, 7, _BM), yidx),
        ],
        out_specs=[
            pl.BlockSpec((1, 1, 1), lambda b, j: (b, 0, 0)),
            pl.BlockSpec((1, 1, 1), lambda b, j: (b, 0, 0)),
        ],
        out_shape=[
            jax.ShapeDtypeStruct((B, 1, 1), jnp.float32),
            jax.ShapeDtypeStruct((B, 1, 1), jnp.float32),
        ],
        scratch_shapes=[
            pltpu.VMEM((N, _BH), jnp.float32),
            pltpu.VMEM((N, _BH), jnp.float32),
            pltpu.VMEM((N, 128), jnp.float32),
            pltpu.VMEM((1, 128), jnp.float32),
        ],
        compiler_params=pltpu.CompilerParams(
            dimension_semantics=("parallel", "arbitrary")),
    )(xpa, ypa)

    return (jnp.sum(rowsums) / (B * N)) + (jnp.sum(colsums) / (B * M))


# speedup vs baseline: 2.9301x; 1.0038x over previous
import functools

import jax
import jax.numpy as jnp
from jax.experimental import pallas as pl
from jax.experimental.pallas import tpu as pltpu

_BM = 1024
_BH = 512


def _dot_dims():
    return (((0,), (0,)), ((), ()))


def _tree_colmin(dg):
    n = dg.shape[0]
    while n > 256:
        n //= 4
        dg = jnp.minimum(jnp.minimum(dg[:n], dg[n:2 * n]),
                         jnp.minimum(dg[2 * n:3 * n], dg[3 * n:]))
    return jnp.min(dg, axis=0, keepdims=True)


def _reduce_half(d):
    folds = []
    csum = None
    for g in range(_BH // 128):
        dg = d[:, g * 128:(g + 1) * 128]
        cm = jnp.maximum(_tree_colmin(dg), 0.0)
        csum = cm if csum is None else csum + cm
        folds.append(dg)
    while len(folds) > 1:
        folds = [jnp.minimum(folds[i], folds[i + 1])
                 for i in range(0, len(folds), 2)]
    return folds[0], csum


def _pair_step(p, j, xpa, ypa, rowsum_ref, colsum_ref,
               z0_ref, z1_ref, rowacc_ref, csum_ref, num_mblocks):
    f1, s1 = _reduce_half(z1_ref[...])

    z0_ref[...] = jax.lax.dot_general(xpa, ypa[:, :_BH], _dot_dims(),
                                      preferred_element_type=jnp.float32)
    f0, s0 = _reduce_half(z0_ref[...])

    acc = jnp.minimum(rowacc_ref[...], jnp.minimum(f0, f1))
    rowacc_ref[...] = jnp.where(j == 0, f0, acc)

    csum_ref[...] = jnp.where(j == 0, s0, csum_ref[...] + s0 + s1)

    z1_ref[...] = jax.lax.dot_general(xpa, ypa[:, _BH:], _dot_dims(),
                                      preferred_element_type=jnp.float32)

    @pl.when(j == num_mblocks - 1)
    def _finish():
        f1b, s1b = _reduce_half(z1_ref[...])
        rowall = jnp.minimum(rowacc_ref[...], f1b)
        rowmin = jnp.min(rowall, axis=1, keepdims=True)
        rowclamped = jnp.maximum(rowmin, 0.0)
        rowsum_ref[p, :, :] = jnp.sum(rowclamped, axis=0, keepdims=True)
        colsum_ref[p, :, :] = jnp.sum(csum_ref[...] + s1b, axis=1,
                                      keepdims=True)


def _chamfer_body(xpa_ref, ypa_ref, rowsum_ref, colsum_ref,
                  z0a_ref, z1a_ref, z0b_ref, z1b_ref,
                  rowacca_ref, rowaccb_ref, csuma_ref, csumb_ref,
                  *, num_mblocks):
    j = pl.program_id(1)
    _pair_step(0, j, xpa_ref[0], ypa_ref[0], rowsum_ref, colsum_ref,
               z0a_ref, z1a_ref, rowacca_ref, csuma_ref, num_mblocks)
    _pair_step(1, j, xpa_ref[1], ypa_ref[1], rowsum_ref, colsum_ref,
               z0b_ref, z1b_ref, rowaccb_ref, csumb_ref, num_mblocks)


def _hi_lo(v):
    hi = jax.lax.bitcast_convert_type(
        jax.lax.bitcast_convert_type(v, jnp.uint32) & jnp.uint32(0xFFFF0000),
        jnp.float32)
    return hi, v - hi


def kernel(x, y):
    B, N, D = x.shape
    M = y.shape[1]
    num_mblocks = M // _BM

    x2 = jnp.sum(x * x, axis=-1)[:, None, :]
    y2 = jnp.sum(y * y, axis=-1)[:, None, :]
    x2hi, x2lo = _hi_lo(x2)
    y2hi, y2lo = _hi_lo(y2)
    ones_n = jnp.ones((B, 1, N), jnp.float32)
    ones_m = jnp.ones((B, 1, M), jnp.float32)
    xpa = jnp.concatenate(
        [jnp.swapaxes(-2.0 * x, 1, 2), x2hi, x2lo, ones_n, ones_n],
        axis=1).astype(jnp.bfloat16)
    ypa = jnp.concatenate(
        [jnp.swapaxes(y, 1, 2), ones_m, ones_m, y2hi, y2lo],
        axis=1).astype(jnp.bfloat16)

    nj = num_mblocks
    yidx = lambda b, j: (b, 0, j)

    rowsums, colsums = pl.pallas_call(
        functools.partial(_chamfer_body, num_mblocks=num_mblocks),
        grid=(B // 2, nj),
        in_specs=[
            pl.BlockSpec((2, 7, N), lambda b, j: (b, 0, 0)),
            pl.BlockSpec((2, 7, _BM), yidx),
        ],
        out_specs=[
            pl.BlockSpec((2, 1, 1), lambda b, j: (b, 0, 0)),
            pl.BlockSpec((2, 1, 1), lambda b, j: (b, 0, 0)),
        ],
        out_shape=[
            jax.ShapeDtypeStruct((B, 1, 1), jnp.float32),
            jax.ShapeDtypeStruct((B, 1, 1), jnp.float32),
        ],
        scratch_shapes=[
            pltpu.VMEM((N, _BH), jnp.float32),
            pltpu.VMEM((N, _BH), jnp.float32),
            pltpu.VMEM((N, _BH), jnp.float32),
            pltpu.VMEM((N, _BH), jnp.float32),
            pltpu.VMEM((N, 128), jnp.float32),
            pltpu.VMEM((N, 128), jnp.float32),
            pltpu.VMEM((1, 128), jnp.float32),
            pltpu.VMEM((1, 128), jnp.float32),
        ],
        compiler_params=pltpu.CompilerParams(
            dimension_semantics=("parallel", "arbitrary")),
    )(xpa, ypa)

    return (jnp.sum(rowsums) / (B * N)) + (jnp.sum(colsums) / (B * M))


# speedup vs baseline: 3.0560x; 1.0430x over previous
import functools

import jax
import jax.numpy as jnp
from jax.experimental import pallas as pl
from jax.experimental.pallas import tpu as pltpu

_BM = 1024
_BH = 512


def _dot_dims():
    return (((0,), (0,)), ((), ()))


def _tree_colmin(dg):
    n = dg.shape[0]
    while n > 256:
        n //= 4
        dg = jnp.minimum(jnp.minimum(dg[:n], dg[n:2 * n]),
                         jnp.minimum(dg[2 * n:3 * n], dg[3 * n:]))
    return jnp.min(dg, axis=0, keepdims=True)


def _reduce_half(d):
    folds = []
    csum = None
    for g in range(_BH // 128):
        dg = d[:, g * 128:(g + 1) * 128]
        cm = jnp.maximum(_tree_colmin(dg), 0.0)
        csum = cm if csum is None else csum + cm
        folds.append(dg)
    while len(folds) > 1:
        folds = [jnp.minimum(folds[i], folds[i + 1])
                 for i in range(0, len(folds), 2)]
    return folds[0], csum


def _chamfer_body(xpa_ref, ypa_ref, rowsum_ref, colsum_ref,
                  z0a_ref, z1a_ref, z0b_ref, z1b_ref,
                  rowacca_ref, rowaccb_ref, csuma_ref, csumb_ref,
                  *, num_mblocks):
    j = pl.program_id(1)
    xa, ya = xpa_ref[0], ypa_ref[0]
    xb, yb = xpa_ref[1], ypa_ref[1]

    f1a, s1a = _reduce_half(z1a_ref[...])
    f1b, s1b = _reduce_half(z1b_ref[...])

    z0a_ref[...] = jax.lax.dot_general(xa, ya[:, :_BH], _dot_dims(),
                                       preferred_element_type=jnp.float32)
    z0b_ref[...] = jax.lax.dot_general(xb, yb[:, :_BH], _dot_dims(),
                                       preferred_element_type=jnp.float32)
    z1a_ref[...] = jax.lax.dot_general(xa, ya[:, _BH:], _dot_dims(),
                                       preferred_element_type=jnp.float32)
    z1b_ref[...] = jax.lax.dot_general(xb, yb[:, _BH:], _dot_dims(),
                                       preferred_element_type=jnp.float32)

    f0a, s0a = _reduce_half(z0a_ref[...])
    f0b, s0b = _reduce_half(z0b_ref[...])

    acca = jnp.minimum(rowacca_ref[...], jnp.minimum(f0a, f1a))
    rowacca_ref[...] = jnp.where(j == 0, f0a, acca)
    csuma_ref[...] = jnp.where(j == 0, s0a, csuma_ref[...] + s0a + s1a)
    accb = jnp.minimum(rowaccb_ref[...], jnp.minimum(f0b, f1b))
    rowaccb_ref[...] = jnp.where(j == 0, f0b, accb)
    csumb_ref[...] = jnp.where(j == 0, s0b, csumb_ref[...] + s0b + s1b)

    @pl.when(j == num_mblocks - 1)
    def _finish():
        for p, z1_ref, rowacc_ref, csum_ref in (
                (0, z1a_ref, rowacca_ref, csuma_ref),
                (1, z1b_ref, rowaccb_ref, csumb_ref)):
            f1c, s1c = _reduce_half(z1_ref[...])
            rowall = jnp.minimum(rowacc_ref[...], f1c)
            rowmin = jnp.min(rowall, axis=1, keepdims=True)
            rowclamped = jnp.maximum(rowmin, 0.0)
            rowsum_ref[p, :, :] = jnp.sum(rowclamped, axis=0, keepdims=True)
            colsum_ref[p, :, :] = jnp.sum(csum_ref[...] + s1c, axis=1,
                                          keepdims=True)


def _hi_lo(v):
    hi = jax.lax.bitcast_convert_type(
        jax.lax.bitcast_convert_type(v, jnp.uint32) & jnp.uint32(0xFFFF0000),
        jnp.float32)
    return hi, v - hi


def kernel(x, y):
    B, N, D = x.shape
    M = y.shape[1]
    num_mblocks = M // _BM

    x2 = jnp.sum(x * x, axis=-1)[:, None, :]
    y2 = jnp.sum(y * y, axis=-1)[:, None, :]
    x2hi, x2lo = _hi_lo(x2)
    y2hi, y2lo = _hi_lo(y2)
    ones_n = jnp.ones((B, 1, N), jnp.float32)
    ones_m = jnp.ones((B, 1, M), jnp.float32)
    xpa = jnp.concatenate(
        [jnp.swapaxes(-2.0 * x, 1, 2), x2hi, x2lo, ones_n, ones_n],
        axis=1).astype(jnp.bfloat16)
    ypa = jnp.concatenate(
        [jnp.swapaxes(y, 1, 2), ones_m, ones_m, y2hi, y2lo],
        axis=1).astype(jnp.bfloat16)

    nj = num_mblocks
    yidx = lambda b, j: (b, 0, j)

    rowsums, colsums = pl.pallas_call(
        functools.partial(_chamfer_body, num_mblocks=num_mblocks),
        grid=(B // 2, nj),
        in_specs=[
            pl.BlockSpec((2, 7, N), lambda b, j: (b, 0, 0)),
            pl.BlockSpec((2, 7, _BM), yidx),
        ],
        out_specs=[
            pl.BlockSpec((2, 1, 1), lambda b, j: (b, 0, 0)),
            pl.BlockSpec((2, 1, 1), lambda b, j: (b, 0, 0)),
        ],
        out_shape=[
            jax.ShapeDtypeStruct((B, 1, 1), jnp.float32),
            jax.ShapeDtypeStruct((B, 1, 1), jnp.float32),
        ],
        scratch_shapes=[
            pltpu.VMEM((N, _BH), jnp.float32),
            pltpu.VMEM((N, _BH), jnp.float32),
            pltpu.VMEM((N, _BH), jnp.float32),
            pltpu.VMEM((N, _BH), jnp.float32),
            pltpu.VMEM((N, 128), jnp.float32),
            pltpu.VMEM((N, 128), jnp.float32),
            pltpu.VMEM((1, 128), jnp.float32),
            pltpu.VMEM((1, 128), jnp.float32),
        ],
        compiler_params=pltpu.CompilerParams(
            dimension_semantics=("parallel", "arbitrary")),
    )(xpa, ypa)

    return (jnp.sum(rowsums) / (B * N)) + (jnp.sum(colsums) / (B * M))
